# baseline (speedup 1.0000x reference)
import functools

import jax
import jax.numpy as jnp
from jax import lax
from jax.experimental import pallas as pl
from jax.experimental.pallas import tpu as pltpu
from jax.experimental.pallas import tpu_sc as plsc

_NC = 2
_NS = 16
_NW = _NC * _NS
_LANES = 16


def _make_sc_gather(B, N, H):
  b_per_w = B // _NW
  chunks_per_row = H // _LANES
  mesh = plsc.VectorSubcoreMesh(core_axis_name="c", subcore_axis_name="s")
  out = jax.ShapeDtypeStruct((B, H), jnp.float32)

  @functools.partial(
      pl.kernel,
      mesh=mesh,
      out_type=(out, out, out),
      compiler_params=pltpu.CompilerParams(use_tc_tiling_on_sc=False),
      scratch_types=[
          pltpu.VMEM((b_per_w,), jnp.int32),
          pltpu.VMEM((b_per_w, H), jnp.float32),
          pltpu.VMEM((b_per_w, H), jnp.float32),
          pltpu.VMEM((b_per_w, H), jnp.float32),
          pltpu.SemaphoreType.DMA,
          pltpu.SemaphoreType.DMA,
          pltpu.SemaphoreType.DMA,
      ],
  )
  def sc_gather(idx_hbm, mu_hbm, sp_hbm, mu_out, l_out, sig_out,
                idx_v, mu_v, l_v, s_v, sem0, sem1, sem2):
    wid = lax.axis_index("s") * _NC + lax.axis_index("c")
    base = wid * b_per_w
    sl = pl.ds(base, b_per_w)

    pltpu.sync_copy(idx_hbm.at[sl], idx_v)
    cp_mu = pltpu.async_copy(mu_hbm.at[idx_v], mu_v, sem0)
    cp_l = pltpu.async_copy(sp_hbm.at[idx_v], l_v, sem1)

    cp_l.wait()
    cp_lout = pltpu.async_copy(l_v, l_out.at[sl], sem2)

    def body(r, carry):
      for j in range(chunks_per_row):
        x = l_v[r, pl.ds(j * _LANES, _LANES)]
        s_v[r, pl.ds(j * _LANES, _LANES)] = x * x
      return carry

    lax.fori_loop(0, b_per_w, body, 0)

    cp_mu.wait()
    cp_muout = pltpu.async_copy(mu_v, mu_out.at[sl], sem0)
    cp_sout = pltpu.async_copy(s_v, sig_out.at[sl], sem1)
    cp_lout.wait()
    cp_muout.wait()
    cp_sout.wait()

  return sc_gather


def kernel(X, indexes, mu, Sigma_param):
  del X
  B = indexes.shape[0]
  N, H = mu.shape
  idx = indexes.astype(jnp.int32)
  mu_g, L, Sigma = _make_sc_gather(B, N, H)(idx, mu, Sigma_param)
  return (mu_g, L, Sigma)


# speedup vs baseline: 2.2554x; 2.2554x over previous
import functools

import jax
import jax.numpy as jnp
from jax import lax
from jax.experimental import pallas as pl
from jax.experimental.pallas import tpu as pltpu
from jax.experimental.pallas import tpu_sc as plsc

_NC = 2
_NS = 16
_NW = _NC * _NS
_L = 16
_TW = 128


def _make_sc_gather(B, N, H):
  b_per_w = B // _NW
  n_tc = (N + _TW - 1) // _TW
  n_chunks = b_per_w // _L
  mesh = plsc.VectorSubcoreMesh(core_axis_name="c", subcore_axis_name="s")

  @functools.partial(
      pl.kernel,
      mesh=mesh,
      out_type=jax.ShapeDtypeStruct((B, 4 * H), jnp.float32),
      compiler_params=pltpu.CompilerParams(
          use_tc_tiling_on_sc=True,
          disable_bounds_checks=True,
          needs_layout_passes=False,
      ),
      scratch_types=[
          pltpu.VMEM((b_per_w + _L,), jnp.int32),
          pltpu.VMEM((2, 2, H, _TW), jnp.float32),
          pltpu.VMEM((2, 2, H, _TW), jnp.float32),
          pltpu.VMEM((2 * _L, 4 * H), jnp.float32),
          pltpu.SemaphoreType.DMA,
          pltpu.SemaphoreType.DMA,
          pltpu.SemaphoreType.DMA,
      ],
  )
  def sc_gather(idx_hbm, mu_hbm, sp_hbm, fused_out,
                idx_v, blk_mu, blk_sp, rowbuf, semf0, semf1, semo):
    wid = lax.axis_index("s") * _NC + lax.axis_index("c")
    base = wid * b_per_w
    iotas = [
        lax.broadcasted_iota(jnp.int32, (_L,), 0) + j * _L
        for j in range(H // _L)
    ]
    zeros16 = jnp.zeros((_L,), jnp.int32)
    fsems = (semf0, semf1)

    pltpu.sync_copy(idx_hbm.at[pl.ds(base, b_per_w)],
                    idx_v.at[pl.ds(0, b_per_w)])

    def fire(ph, pp, s):
      tc = jnp.minimum(lax.shift_right_logical(s, 7), n_tc - 1)
      off = pl.multiple_of(tc * _TW, _TW)
      pltpu.async_copy(mu_hbm.at[:, pl.ds(off, _TW)], blk_mu.at[ph, pp],
                       fsems[ph])
      pltpu.async_copy(sp_hbm.at[:, pl.ds(off, _TW)], blk_sp.at[ph, pp],
                       fsems[ph])

    def extract(ph, pp, s, rowpos):
      rr = jnp.bitwise_and(s, _TW - 1)
      ri = zeros16 + rr
      for j in range(H // _L):
        m = plsc.load_gather(blk_mu.at[ph, pp], [iotas[j], ri])
        p = plsc.load_gather(blk_sp.at[ph, pp], [iotas[j], ri])
        rowbuf[rowpos, pl.ds(j * _L, _L)] = m
        rowbuf[rowpos, pl.ds(H + j * _L, _L)] = p
        rowbuf[rowpos, pl.ds(2 * H + j * _L, _L)] = p * p

    v0 = idx_v[pl.ds(0, _L)]
    fire(0, 0, v0[0])
    fire(0, 1, v0[1])

    def body(c, carry):
      v = idx_v[pl.ds(c * _L, _L)]
      vn = idx_v[pl.ds(c * _L + _L, _L)]
      rbase = jnp.bitwise_and(c, 1) * _L
      for g in range(8):
        ph = g & 1
        nph = (g + 1) & 1
        if g < 7:
          fire(nph, 0, v[2 * g + 2])
          fire(nph, 1, v[2 * g + 3])
        else:
          @pl.when(c < n_chunks - 1)
          def _():
            fire(nph, 0, vn[0])
            fire(nph, 1, vn[1])
        pltpu.make_async_copy(
            mu_hbm.at[:, pl.ds(0, _TW)], blk_mu.at[ph, 0], fsems[ph]).wait()
        pltpu.make_async_copy(
            mu_hbm.at[:, pl.ds(0, _TW)], blk_mu.at[ph, 1], fsems[ph]).wait()
        pltpu.make_async_copy(
            mu_hbm.at[:, pl.ds(0, _TW)], blk_sp.at[ph, 0], fsems[ph]).wait()
        pltpu.make_async_copy(
            mu_hbm.at[:, pl.ds(0, _TW)], blk_sp.at[ph, 1], fsems[ph]).wait()
        extract(ph, 0, v[2 * g], rbase + 2 * g)
        extract(ph, 1, v[2 * g + 1], rbase + 2 * g + 1)

      @pl.when(jnp.bitwise_and(c, 1) == 1)
      def _():
        off = pl.multiple_of(base + (c - 1) * _L, 2 * _L)
        cp = pltpu.async_copy(rowbuf, fused_out.at[pl.ds(off, 2 * _L)], semo)
        cp.wait()

      return carry

    lax.fori_loop(0, n_chunks, body, 0)

  return sc_gather


def kernel(X, indexes, mu, Sigma_param):
  del X
  B = indexes.shape[0]
  N, H = mu.shape
  idx = indexes.astype(jnp.int32)
  fused = _make_sc_gather(B, N, H)(idx, mu.T, Sigma_param.T)
  return (fused[:, :H], fused[:, H:2 * H], fused[:, 2 * H:3 * H])
